# baseline (speedup 1.0000x reference)
import jax
import jax.numpy as jnp
from jax import lax
from jax.experimental import pallas as pl
from jax.experimental.pallas import tpu as pltpu
from jax.experimental.pallas import tpu_sc as plsc

MAXLEN = 200
EMBED_DIM = 64
PAD_DIM = 128
LANES = 16

NUM_WORKERS = 32
SEQ_PER_WORKER = 128
CHUNK_ROWS = 200
CHUNKS = SEQ_PER_WORKER
SUB = 100
NSUB = CHUNK_ROWS // SUB
GRP = 4
GROUPS = CHUNKS // GRP
IDX_GRP_ROWS = GRP * NSUB


def _emb_body(x_hbm, tok_hbm, pos_hbm, out_hbm, idx_v, rows_v, pos_v,
              gsems, osems, isems):
    wid = lax.axis_index("s") * 2 + lax.axis_index("c")
    base_row = wid * (SEQ_PER_WORKER * MAXLEN)
    idx_base = wid * (SEQ_PER_WORKER * MAXLEN // SUB)

    def idx_copy(g):
        gb = g & 1
        return pltpu.make_async_copy(
            x_hbm.at[pl.ds(pl.multiple_of(idx_base + g * IDX_GRP_ROWS, 8),
                           IDX_GRP_ROWS)],
            idx_v.at[pl.ds(gb * IDX_GRP_ROWS, IDX_GRP_ROWS)],
            isems.at[gb])

    def gather(c, b, issue):
        for i in range(NSUB):
            row = ((c // GRP) & 1) * IDX_GRP_ROWS + (c % GRP) * NSUB + i
            d = pltpu.make_async_copy(
                tok_hbm.at[idx_v.at[row]],
                rows_v.at[b, pl.ds(i * SUB, SUB)],
                gsems.at[b])
            d.start() if issue else d.wait()

    def out_write(c, b, issue):
        d = pltpu.make_async_copy(
            rows_v.at[b],
            out_hbm.at[pl.ds(pl.multiple_of(base_row + c * CHUNK_ROWS, 8),
                             CHUNK_ROWS)],
            osems.at[b])
        d.start() if issue else d.wait()

    pltpu.sync_copy(pos_hbm, pos_v)
    d0 = idx_copy(0)
    d0.start()
    d0.wait()
    gather(0, 0, True)
    idx_copy(1).start()

    def group_body(g, carry):
        for k in range(GRP):
            b = k & 1
            c = GRP * g + k

            @pl.when(c >= 1)
            def _():
                out_write(c - 1, 1 - b, False)

            @pl.when(c + 1 < CHUNKS)
            def _():
                if k == GRP - 1:
                    idx_copy(g + 1).wait()
                gather(c + 1, 1 - b, True)

            gather(c, b, False)
            if k == GRP - 1:
                @pl.when(g + 2 < GROUPS)
                def _():
                    idx_copy(g + 2).start()

            def row_body(r, carry2, _b=b):
                for j in range(EMBED_DIM // LANES):
                    pv = pos_v[pl.ds(r * EMBED_DIM + j * LANES, LANES)]
                    plsc.addupdate(
                        rows_v.at[_b, r, pl.ds(j * LANES, LANES)], pv)
                return carry2

            lax.fori_loop(0, CHUNK_ROWS, row_body, None)
            out_write(c, b, True)
        return carry

    lax.fori_loop(0, GROUPS, group_body, None)
    out_write(CHUNKS - 1, 1, False)


def _transpose_pad_body(xt_ref, o_ref):
    o_ref[:, :EMBED_DIM] = xt_ref[...].T
    o_ref[:, EMBED_DIM:] = jnp.zeros_like(o_ref[:, EMBED_DIM:])


def _transpose_pad_table(tok_t):
    _, vocab = tok_t.shape
    blk = 7936
    return pl.pallas_call(
        _transpose_pad_body,
        grid=(pl.cdiv(vocab, blk),),
        in_specs=[pl.BlockSpec((EMBED_DIM, blk), lambda i: (0, i))],
        out_specs=pl.BlockSpec((blk, PAD_DIM), lambda i: (i, 0)),
        out_shape=jax.ShapeDtypeStruct((vocab, PAD_DIM), jnp.float32),
    )(tok_t)


def kernel(x, token_table, pos_table):
    batch, maxlen = x.shape
    _, embed_dim = token_table.shape
    n_rows = batch * maxlen
    x2 = x.reshape(n_rows // SUB, SUB).astype(jnp.int32)
    tok_pad = _transpose_pad_table(token_table.T)
    pos_flat = pos_table.reshape(-1)

    call = pl.kernel(
        _emb_body,
        out_type=jax.ShapeDtypeStruct((n_rows, PAD_DIM), jnp.float32),
        mesh=plsc.VectorSubcoreMesh(core_axis_name="c", subcore_axis_name="s"),
        scratch_types=[
            pltpu.VMEM((2 * IDX_GRP_ROWS, SUB), jnp.int32),
            pltpu.VMEM((2, CHUNK_ROWS, PAD_DIM), jnp.float32),
            pltpu.VMEM((MAXLEN * EMBED_DIM,), jnp.float32),
            pltpu.SemaphoreType.DMA((2,)),
            pltpu.SemaphoreType.DMA((2,)),
            pltpu.SemaphoreType.DMA((2,)),
        ],
        compiler_params=pltpu.CompilerParams(use_tc_tiling_on_sc=True),
    )
    out_flat = call(x2, tok_pad, pos_flat)
    return out_flat[:, :embed_dim].reshape(batch, maxlen, embed_dim)


# speedup vs baseline: 1.0045x; 1.0045x over previous
import jax
import jax.numpy as jnp
from jax import lax
from jax.experimental import pallas as pl
from jax.experimental.pallas import tpu as pltpu
from jax.experimental.pallas import tpu_sc as plsc

MAXLEN = 200
EMBED_DIM = 64
PAD_DIM = 128
LANES = 16

NUM_WORKERS = 32
SEQ_PER_WORKER = 128
CHUNK_ROWS = 200
CHUNKS = SEQ_PER_WORKER
SUB = 100
NSUB = CHUNK_ROWS // SUB
GRP = 4
GROUPS = CHUNKS // GRP
IDX_GRP_ROWS = GRP * NSUB


def _emb_body(x_hbm, tok_hbm, pos_hbm, out_hbm, idx_v, rows_v, pos_v,
              gsems, osems, isems):
    wid = lax.axis_index("s") * 2 + lax.axis_index("c")
    base_row = wid * (SEQ_PER_WORKER * MAXLEN)
    idx_base = wid * (SEQ_PER_WORKER * MAXLEN // SUB)

    def idx_copy(g):
        gb = g & 1
        return pltpu.make_async_copy(
            x_hbm.at[pl.ds(pl.multiple_of(idx_base + g * IDX_GRP_ROWS, 8),
                           IDX_GRP_ROWS)],
            idx_v.at[pl.ds(gb * IDX_GRP_ROWS, IDX_GRP_ROWS)],
            isems.at[gb])

    def gather(c, b, issue):
        for i in range(NSUB):
            row = ((c // GRP) & 1) * IDX_GRP_ROWS + (c % GRP) * NSUB + i
            d = pltpu.make_async_copy(
                tok_hbm.at[idx_v.at[row]],
                rows_v.at[b, pl.ds(i * SUB, SUB)],
                gsems.at[b])
            d.start() if issue else d.wait()

    def out_write(c, b, issue):
        d = pltpu.make_async_copy(
            rows_v.at[b],
            out_hbm.at[pl.ds(pl.multiple_of(base_row + c * CHUNK_ROWS, 8),
                             CHUNK_ROWS)],
            osems.at[b])
        d.start() if issue else d.wait()

    pltpu.sync_copy(pos_hbm, pos_v)
    d0 = idx_copy(0)
    d0.start()
    d0.wait()
    gather(0, 0, True)
    idx_copy(1).start()

    def group_body(g, carry):
        for k in range(GRP):
            b = k & 1
            c = GRP * g + k

            gather(c, b, False)
            if k == GRP - 1:
                @pl.when(g + 2 < GROUPS)
                def _():
                    idx_copy(g + 2).start()

            @pl.when(c >= 1)
            def _():
                out_write(c - 1, 1 - b, False)

            @pl.when(c + 1 < CHUNKS)
            def _():
                if k == GRP - 1:
                    idx_copy(g + 1).wait()
                gather(c + 1, 1 - b, True)

            def row_body(r, carry2, _b=b):
                for j in range(EMBED_DIM // LANES):
                    pv = pos_v[pl.ds(r * EMBED_DIM + j * LANES, LANES)]
                    plsc.addupdate(
                        rows_v.at[_b, r, pl.ds(j * LANES, LANES)], pv)
                return carry2

            lax.fori_loop(0, CHUNK_ROWS, row_body, None)
            out_write(c, b, True)
        return carry

    lax.fori_loop(0, GROUPS, group_body, None)
    out_write(CHUNKS - 1, 1, False)


def _transpose_pad_body(xt_ref, o_ref):
    o_ref[:, :EMBED_DIM] = xt_ref[...].T
    o_ref[:, EMBED_DIM:] = jnp.zeros_like(o_ref[:, EMBED_DIM:])


def _transpose_pad_table(tok_t):
    _, vocab = tok_t.shape
    blk = 7936
    return pl.pallas_call(
        _transpose_pad_body,
        grid=(pl.cdiv(vocab, blk),),
        in_specs=[pl.BlockSpec((EMBED_DIM, blk), lambda i: (0, i))],
        out_specs=pl.BlockSpec((blk, PAD_DIM), lambda i: (i, 0)),
        out_shape=jax.ShapeDtypeStruct((vocab, PAD_DIM), jnp.float32),
    )(tok_t)


def kernel(x, token_table, pos_table):
    batch, maxlen = x.shape
    _, embed_dim = token_table.shape
    n_rows = batch * maxlen
    x2 = x.reshape(n_rows // SUB, SUB).astype(jnp.int32)
    tok_pad = _transpose_pad_table(token_table.T)
    pos_flat = pos_table.reshape(-1)

    call = pl.kernel(
        _emb_body,
        out_type=jax.ShapeDtypeStruct((n_rows, PAD_DIM), jnp.float32),
        mesh=plsc.VectorSubcoreMesh(core_axis_name="c", subcore_axis_name="s"),
        scratch_types=[
            pltpu.VMEM((2 * IDX_GRP_ROWS, SUB), jnp.int32),
            pltpu.VMEM((2, CHUNK_ROWS, PAD_DIM), jnp.float32),
            pltpu.VMEM((MAXLEN * EMBED_DIM,), jnp.float32),
            pltpu.SemaphoreType.DMA((2,)),
            pltpu.SemaphoreType.DMA((2,)),
            pltpu.SemaphoreType.DMA((2,)),
        ],
        compiler_params=pltpu.CompilerParams(use_tc_tiling_on_sc=True),
    )
    out_flat = call(x2, tok_pad, pos_flat)
    return out_flat[:, :embed_dim].reshape(batch, maxlen, embed_dim)


# speedup vs baseline: 1.2630x; 1.2573x over previous
import jax
import jax.numpy as jnp
from jax import lax
from jax.experimental import pallas as pl
from jax.experimental.pallas import tpu as pltpu
from jax.experimental.pallas import tpu_sc as plsc

MAXLEN = 200
EMBED_DIM = 64
PAD_DIM = 128
LANES = 16

NUM_WORKERS = 32
SEQ_PER_WORKER = 128
CHUNK_ROWS = 200
CHUNKS = SEQ_PER_WORKER
SUB = 100
NSUB = CHUNK_ROWS // SUB
GRP = 4
GROUPS = CHUNKS // GRP
IDX_GRP_ROWS = GRP * NSUB


def _emb_body(x_hbm, tok_hbm, pos_hbm, out_hbm, idx_v, rows_v, pos_v,
              gsems, osems, isems):
    wid = lax.axis_index("s") * 2 + lax.axis_index("c")
    base_row = wid * (SEQ_PER_WORKER * MAXLEN)
    idx_base = wid * (SEQ_PER_WORKER * MAXLEN // SUB)

    def idx_copy(g):
        gb = g & 1
        return pltpu.make_async_copy(
            x_hbm.at[pl.ds(pl.multiple_of(idx_base + g * IDX_GRP_ROWS, 8),
                           IDX_GRP_ROWS)],
            idx_v.at[pl.ds(gb * IDX_GRP_ROWS, IDX_GRP_ROWS)],
            isems.at[gb])

    def gather(c, b, issue):
        for i in range(NSUB):
            row = ((c // GRP) & 1) * IDX_GRP_ROWS + (c % GRP) * NSUB + i
            d = pltpu.make_async_copy(
                tok_hbm.at[idx_v.at[row]],
                rows_v.at[b, pl.ds(i * SUB, SUB)],
                gsems.at[b])
            d.start() if issue else d.wait()

    def out_write(c, b, issue):
        d = pltpu.make_async_copy(
            rows_v.at[b],
            out_hbm.at[pl.ds(pl.multiple_of(base_row + c * CHUNK_ROWS, 8),
                             CHUNK_ROWS)],
            osems.at[b])
        d.start() if issue else d.wait()

    pltpu.sync_copy(pos_hbm, pos_v)
    d0 = idx_copy(0)
    d0.start()
    d0.wait()
    gather(0, 0, True)
    idx_copy(1).start()

    def group_body(g, carry):
        for k in range(GRP):
            b = k & 1
            c = GRP * g + k

            gather(c, b, False)
            if k == GRP - 1:
                @pl.when(g + 2 < GROUPS)
                def _():
                    idx_copy(g + 2).start()

            @pl.when(c >= 1)
            def _():
                out_write(c - 1, 1 - b, False)

            @pl.when(c + 1 < CHUNKS)
            def _():
                if k == GRP - 1:
                    idx_copy(g + 1).wait()
                gather(c + 1, 1 - b, True)

            def row_body(r, carry2, _b=b):
                for j in range(EMBED_DIM // LANES):
                    pv = pos_v[r, pl.ds(j * LANES, LANES)]
                    plsc.addupdate(
                        rows_v.at[_b, r, pl.ds(j * LANES, LANES)], pv)
                return carry2

            lax.fori_loop(0, CHUNK_ROWS, row_body, None)
            out_write(c, b, True)
        return carry

    lax.fori_loop(0, GROUPS, group_body, None)
    out_write(CHUNKS - 1, 1, False)


def _transpose_pad_body(xt_ref, o_ref):
    o_ref[:, :EMBED_DIM] = xt_ref[...].T
    o_ref[:, EMBED_DIM:] = jnp.zeros_like(o_ref[:, EMBED_DIM:])


def _transpose_pad_table(tok_t):
    _, vocab = tok_t.shape
    blk = 7936
    return pl.pallas_call(
        _transpose_pad_body,
        grid=(pl.cdiv(vocab, blk),),
        in_specs=[pl.BlockSpec((EMBED_DIM, blk), lambda i: (0, i))],
        out_specs=pl.BlockSpec((blk, PAD_DIM), lambda i: (i, 0)),
        out_shape=jax.ShapeDtypeStruct((vocab, PAD_DIM), jnp.float32),
    )(tok_t)


def kernel(x, token_table, pos_table):
    batch, maxlen = x.shape
    _, embed_dim = token_table.shape
    n_rows = batch * maxlen
    x2 = x.reshape(n_rows // SUB, SUB).astype(jnp.int32)
    tok_pad = _transpose_pad_table(token_table.T)

    call = pl.kernel(
        _emb_body,
        out_type=jax.ShapeDtypeStruct((n_rows, PAD_DIM), jnp.float32),
        mesh=plsc.VectorSubcoreMesh(core_axis_name="c", subcore_axis_name="s"),
        scratch_types=[
            pltpu.VMEM((2 * IDX_GRP_ROWS, SUB), jnp.int32),
            pltpu.VMEM((2, CHUNK_ROWS, PAD_DIM), jnp.float32),
            pltpu.VMEM((MAXLEN, EMBED_DIM), jnp.float32),
            pltpu.SemaphoreType.DMA((2,)),
            pltpu.SemaphoreType.DMA((2,)),
            pltpu.SemaphoreType.DMA((2,)),
        ],
        compiler_params=pltpu.CompilerParams(use_tc_tiling_on_sc=True),
    )
    out_flat = call(x2, tok_pad, pos_table)
    return out_flat[:, :embed_dim].reshape(batch, maxlen, embed_dim)
